# baseline (device time: 74186 ns/iter reference)
import jax
import jax.numpy as jnp
from jax import lax
from jax.experimental import pallas as pl
from jax.experimental.pallas import tpu as pltpu

N_DEV = 4


def kernel(x, w_mat, scale_x, scale_w):
    x = x.astype(jnp.bfloat16)
    w_mat = w_mat.astype(jnp.bfloat16)
    m_per, k = x.shape
    _, n = w_mat.shape
    n_per = n // N_DEV

    def body(x_ref, w_ref, sx_ref, sw_ref, out_ref, ybuf, send_sems, recv_sems):
        my = lax.axis_index("i")

        barrier_sem = pltpu.get_barrier_semaphore()
        for s in range(1, N_DEV):
            pl.semaphore_signal(
                barrier_sem, inc=1,
                device_id=((my + s) % N_DEV,),
                device_id_type=pl.DeviceIdType.MESH,
            )
        pl.semaphore_wait(barrier_sem, N_DEV - 1)

        scale = sx_ref[0] * sw_ref[0]

        rdmas = {}
        for s in (2, 1, 3):
            tgt = (my + s) % N_DEV
            blk = jnp.dot(
                x_ref[...], w_ref[:, pl.ds(tgt * n_per, n_per)],
                preferred_element_type=jnp.float32,
            )
            ybuf[:, pl.ds(tgt * n_per, n_per)] = jnp.maximum(
                blk * scale, 0.0
            ).astype(jnp.bfloat16)
            rdma = pltpu.make_async_remote_copy(
                src_ref=ybuf.at[:, pl.ds(tgt * n_per, n_per)],
                dst_ref=out_ref.at[pl.ds(my * m_per, m_per), :],
                send_sem=send_sems.at[s],
                recv_sem=recv_sems.at[s],
                device_id=(tgt,),
                device_id_type=pl.DeviceIdType.MESH,
            )
            rdma.start()
            rdmas[s] = rdma

        blk = jnp.dot(
            x_ref[...], w_ref[:, pl.ds(my * n_per, n_per)],
            preferred_element_type=jnp.float32,
        )
        out_ref[pl.ds(my * m_per, m_per), :] = jnp.maximum(
            blk * scale, 0.0
        ).astype(jnp.bfloat16)

        for s in (1, 3, 2):
            rdmas[s].wait_recv()
        for s in (1, 2, 3):
            rdmas[s].wait_send()

    return pl.pallas_call(
        body,
        out_shape=jax.ShapeDtypeStruct((N_DEV * m_per, n_per), jnp.bfloat16),
        in_specs=[
            pl.BlockSpec(memory_space=pltpu.VMEM),
            pl.BlockSpec(memory_space=pltpu.VMEM),
            pl.BlockSpec(memory_space=pltpu.SMEM),
            pl.BlockSpec(memory_space=pltpu.SMEM),
        ],
        out_specs=pl.BlockSpec(memory_space=pltpu.VMEM),
        scratch_shapes=[
            pltpu.VMEM((m_per, n), jnp.bfloat16),
            pltpu.SemaphoreType.DMA((N_DEV,)),
            pltpu.SemaphoreType.DMA((N_DEV,)),
        ],
        compiler_params=pltpu.CompilerParams(
            collective_id=0,
            vmem_limit_bytes=52 * 1024 * 1024,
        ),
    )(x, w_mat, scale_x, scale_w)


# device time: 47948 ns/iter; 1.5472x vs baseline; 1.5472x over previous
import jax
import jax.numpy as jnp
from jax import lax
from jax.experimental import pallas as pl
from jax.experimental.pallas import tpu as pltpu

N_DEV = 4


def kernel(x, w_mat, scale_x, scale_w):
    m_per, k = x.shape
    _, n = w_mat.shape
    n_per = n // N_DEV

    def body(x_ref, w_hbm, sx_ref, sw_ref, out_ref, wbuf, ybuf,
             dma_sems, send_sems, recv_sems):
        my = lax.axis_index("i")

        def w_dma(s, slot):
            tgt = (my + s) % N_DEV
            return pltpu.make_async_copy(
                w_hbm.at[:, pl.ds(tgt * n_per, n_per)],
                wbuf.at[slot],
                dma_sems.at[slot],
            )

        order = (2, 1, 3, 0)
        w_dma(order[0], 0).start()
        w_dma(order[1], 1).start()

        barrier_sem = pltpu.get_barrier_semaphore()
        for s in range(1, N_DEV):
            pl.semaphore_signal(
                barrier_sem, inc=1,
                device_id=((my + s) % N_DEV,),
                device_id_type=pl.DeviceIdType.MESH,
            )
        pl.semaphore_wait(barrier_sem, N_DEV - 1)

        scale = sx_ref[0] * sw_ref[0]
        x_v = x_ref[...]

        rdmas = {}
        for idx, s in enumerate(order):
            slot = idx % 2
            tgt = (my + s) % N_DEV
            w_dma(s, slot).wait()
            blk = jnp.dot(x_v, wbuf[slot], preferred_element_type=jnp.float32)
            yblk = jnp.maximum(blk * scale, 0.0).astype(jnp.bfloat16)
            if s == 0:
                out_ref[pl.ds(my * m_per, m_per), :] = yblk
            else:
                ybuf[:, pl.ds(tgt * n_per, n_per)] = yblk
                rdma = pltpu.make_async_remote_copy(
                    src_ref=ybuf.at[:, pl.ds(tgt * n_per, n_per)],
                    dst_ref=out_ref.at[pl.ds(my * m_per, m_per), :],
                    send_sem=send_sems.at[s],
                    recv_sem=recv_sems.at[s],
                    device_id=(tgt,),
                    device_id_type=pl.DeviceIdType.MESH,
                )
                rdma.start()
                rdmas[s] = rdma
            if idx + 2 < len(order):
                w_dma(order[idx + 2], slot).start()

        for s in (1, 3, 2):
            rdmas[s].wait_recv()
        for s in (1, 2, 3):
            rdmas[s].wait_send()

    return pl.pallas_call(
        body,
        out_shape=jax.ShapeDtypeStruct((N_DEV * m_per, n_per), jnp.bfloat16),
        in_specs=[
            pl.BlockSpec(memory_space=pltpu.VMEM),
            pl.BlockSpec(memory_space=pltpu.MemorySpace.HBM),
            pl.BlockSpec(memory_space=pltpu.SMEM),
            pl.BlockSpec(memory_space=pltpu.SMEM),
        ],
        out_specs=pl.BlockSpec(memory_space=pltpu.VMEM),
        scratch_shapes=[
            pltpu.VMEM((2, k, n_per), jnp.float32),
            pltpu.VMEM((m_per, n), jnp.bfloat16),
            pltpu.SemaphoreType.DMA((2,)),
            pltpu.SemaphoreType.DMA((N_DEV,)),
            pltpu.SemaphoreType.DMA((N_DEV,)),
        ],
        compiler_params=pltpu.CompilerParams(
            collective_id=0,
            vmem_limit_bytes=60 * 1024 * 1024,
        ),
    )(x, w_mat, scale_x, scale_w)


# device time: 43297 ns/iter; 1.7134x vs baseline; 1.1074x over previous
import jax
import jax.numpy as jnp
from jax import lax
from jax.experimental import pallas as pl
from jax.experimental.pallas import tpu as pltpu

N_DEV = 4
ORDER = (2, 1, 3, 0)


def kernel(x, w_mat, scale_x, scale_w):
    m_per, k = x.shape
    _, n = w_mat.shape
    n_per = n // N_DEV
    m_half = m_per // 2

    def body(x_hbm, w_hbm, sx_ref, sw_ref, out_ref, xbuf, wbuf, ybuf,
             x_sems, w_sems, send_sems, recv_sems):
        my = lax.axis_index("i")

        def x_dma(h):
            return pltpu.make_async_copy(
                x_hbm.at[pl.ds(h * m_half, m_half), :], xbuf.at[h], x_sems.at[h]
            )

        def w_dma(s, slot):
            tgt = (my + s) % N_DEV
            return pltpu.make_async_copy(
                w_hbm.at[:, pl.ds(tgt * n_per, n_per)], wbuf.at[slot],
                w_sems.at[slot],
            )

        x_dma(0).start()
        w_dma(ORDER[0], 0).start()

        barrier_sem = pltpu.get_barrier_semaphore()
        for s in range(1, N_DEV):
            pl.semaphore_signal(
                barrier_sem, inc=1,
                device_id=((my + s) % N_DEV,),
                device_id_type=pl.DeviceIdType.MESH,
            )
        pl.semaphore_wait(barrier_sem, N_DEV - 1)

        scale = sx_ref[0] * sw_ref[0]
        rdmas = {}

        for idx, s in enumerate(ORDER):
            slot = idx % 2
            tgt = (my + s) % N_DEV
            w_dma(s, slot).wait()
            if idx == 0:
                x_dma(0).wait()
                x_dma(1).start()
                w_dma(ORDER[1], 1).start()
            for h in range(2):
                if idx == 0 and h == 1:
                    x_dma(1).wait()
                blk = jnp.dot(
                    xbuf[h], wbuf[slot], preferred_element_type=jnp.float32
                )
                yblk = jnp.maximum(blk * scale, 0.0).astype(jnp.bfloat16)
                row = pl.ds(h * m_half, m_half)
                if s == 0:
                    out_ref[pl.ds(my * m_per + h * m_half, m_half), :] = yblk
                else:
                    ybuf[row, pl.ds(tgt * n_per, n_per)] = yblk
                    rdma = pltpu.make_async_remote_copy(
                        src_ref=ybuf.at[row, pl.ds(tgt * n_per, n_per)],
                        dst_ref=out_ref.at[
                            pl.ds(my * m_per + h * m_half, m_half), :
                        ],
                        send_sem=send_sems.at[s, h],
                        recv_sem=recv_sems.at[s, h],
                        device_id=(tgt,),
                        device_id_type=pl.DeviceIdType.MESH,
                    )
                    rdma.start()
                    rdmas[(s, h)] = rdma
            if idx + 2 < len(ORDER):
                w_dma(ORDER[idx + 2], slot).start()

        for s in (1, 3, 2):
            for h in range(2):
                rdmas[(s, h)].wait_recv()
        for s in (1, 2, 3):
            for h in range(2):
                rdmas[(s, h)].wait_send()

    return pl.pallas_call(
        body,
        out_shape=jax.ShapeDtypeStruct((N_DEV * m_per, n_per), jnp.bfloat16),
        in_specs=[
            pl.BlockSpec(memory_space=pltpu.MemorySpace.HBM),
            pl.BlockSpec(memory_space=pltpu.MemorySpace.HBM),
            pl.BlockSpec(memory_space=pltpu.SMEM),
            pl.BlockSpec(memory_space=pltpu.SMEM),
        ],
        out_specs=pl.BlockSpec(memory_space=pltpu.VMEM),
        scratch_shapes=[
            pltpu.VMEM((2, m_half, k), jnp.float32),
            pltpu.VMEM((2, k, n_per), jnp.float32),
            pltpu.VMEM((m_per, n), jnp.bfloat16),
            pltpu.SemaphoreType.DMA((2,)),
            pltpu.SemaphoreType.DMA((2,)),
            pltpu.SemaphoreType.DMA((N_DEV, 2)),
            pltpu.SemaphoreType.DMA((N_DEV, 2)),
        ],
        compiler_params=pltpu.CompilerParams(
            collective_id=0,
            vmem_limit_bytes=60 * 1024 * 1024,
        ),
    )(x, w_mat, scale_x, scale_w)
